# Initial kernel scaffold; baseline (speedup 1.0000x reference)
import jax
import jax.numpy as jnp
from jax.experimental import pallas as pl


def kernel(token_ids, embedding_matrix):
    raise NotImplementedError("write your pallas kernel here")


# speedup vs baseline: 1.5730x; 1.5730x over previous
import functools

import jax
import jax.numpy as jnp
from jax import lax
from jax.experimental import pallas as pl
from jax.experimental.pallas import tpu as pltpu
from jax.experimental.pallas import tpu_sc as plsc


def _make_gather(num_rows: int, vocab: int, dim: int):
    info = plsc.get_sparse_core_info()
    nc, ns = info.num_cores, info.num_subcores
    nw = nc * ns
    assert num_rows % nw == 0
    per_w = num_rows // nw
    chunk = 128
    assert per_w % chunk == 0
    n_chunks = per_w // chunk

    mesh = plsc.VectorSubcoreMesh(core_axis_name="c", subcore_axis_name="s")

    @functools.partial(
        pl.kernel,
        mesh=mesh,
        compiler_params=pltpu.CompilerParams(use_tc_tiling_on_sc=False),
        out_type=jax.ShapeDtypeStruct((num_rows, dim), jnp.float32),
        scratch_types=[
            pltpu.VMEM((chunk,), jnp.int32),
            pltpu.VMEM((chunk, dim), jnp.float32),
            pltpu.SemaphoreType.DMA,
        ],
    )
    def emb(idx_hbm, tab_hbm, out_hbm, idx_v, rows_v, sem):
        wid = lax.axis_index("s") * nc + lax.axis_index("c")
        base = wid * per_w

        def body(i, carry):
            off = base + i * chunk
            pltpu.sync_copy(idx_hbm.at[pl.ds(off, chunk)], idx_v)
            pltpu.async_copy(tab_hbm.at[idx_v], rows_v, sem).wait()
            pltpu.sync_copy(rows_v, out_hbm.at[pl.ds(off, chunk)])
            return carry

        lax.fori_loop(0, n_chunks, body, 0)

    return emb


def kernel(token_ids, embedding_matrix):
    b, h = token_ids.shape
    v, d = embedding_matrix.shape
    flat = token_ids.reshape(b * h).astype(jnp.int32)
    emb = _make_gather(b * h, v, d)
    out = emb(flat, embedding_matrix)
    return out.reshape(b, h, d)


# speedup vs baseline: 1.8692x; 1.1883x over previous
import functools

import jax
import jax.numpy as jnp
from jax import lax
from jax.experimental import pallas as pl
from jax.experimental.pallas import tpu as pltpu
from jax.experimental.pallas import tpu_sc as plsc

_CHUNK = 128
_GATHERS = 4
_SUPER = _CHUNK * _GATHERS
_NBUF = 2


def _make_gather(num_rows: int, vocab: int, dim: int):
    info = plsc.get_sparse_core_info()
    nc, ns = info.num_cores, info.num_subcores
    nw = nc * ns
    assert num_rows % (nw * _SUPER * _NBUF) == 0
    per_w = num_rows // nw
    n_chunks = per_w // _CHUNK
    n_outer = per_w // (_SUPER * _NBUF)

    mesh = plsc.VectorSubcoreMesh(core_axis_name="c", subcore_axis_name="s")

    @functools.partial(
        pl.kernel,
        mesh=mesh,
        compiler_params=pltpu.CompilerParams(use_tc_tiling_on_sc=False),
        out_type=jax.ShapeDtypeStruct((num_rows, dim), jnp.float32),
        scratch_types=[
            pltpu.VMEM((n_chunks, _CHUNK), jnp.int32),
            pltpu.VMEM((_NBUF, _SUPER, dim), jnp.float32),
            pltpu.SemaphoreType.DMA,
            pltpu.SemaphoreType.DMA,
            pltpu.SemaphoreType.DMA,
        ],
    )
    def emb(idx_hbm, tab_hbm, out_hbm, idx_v, rows_v, gsem, wsem0, wsem1):
        wid = lax.axis_index("s") * nc + lax.axis_index("c")
        base = wid * per_w
        wsems = (wsem0, wsem1)

        pltpu.sync_copy(idx_hbm.at[wid], idx_v)

        def outer(g, carry):
            for b in range(_NBUF):
                sc = g * _NBUF + b
                buf = rows_v.at[b]

                @pl.when(g > 0)
                def _wait_wb():
                    pltpu.make_async_copy(
                        buf, out_hbm.at[pl.ds(base, _SUPER)], wsems[b]
                    ).wait()

                copies = []
                for j in range(_GATHERS):
                    copies.append(pltpu.async_copy(
                        tab_hbm.at[idx_v.at[sc * _GATHERS + j]],
                        buf.at[pl.ds(j * _CHUNK, _CHUNK)],
                        gsem,
                    ))
                for c in copies:
                    c.wait()

                pltpu.make_async_copy(
                    buf, out_hbm.at[pl.ds(base + sc * _SUPER, _SUPER)], wsems[b]
                ).start()
            return carry

        lax.fori_loop(0, n_outer, outer, 0)

        for b in range(_NBUF):
            pltpu.make_async_copy(
                rows_v.at[b], out_hbm.at[pl.ds(base, _SUPER)], wsems[b]
            ).wait()

    return emb


def kernel(token_ids, embedding_matrix):
    b, h = token_ids.shape
    v, d = embedding_matrix.shape
    info = plsc.get_sparse_core_info()
    nw = info.num_cores * info.num_subcores
    flat = token_ids.reshape(nw, (b * h) // (nw * _CHUNK), _CHUNK).astype(jnp.int32)
    emb = _make_gather(b * h, v, d)
    out = emb(flat, embedding_matrix)
    return out.reshape(b, h, d)


# speedup vs baseline: 1.8735x; 1.0023x over previous
import functools

import jax
import jax.numpy as jnp
from jax import lax
from jax.experimental import pallas as pl
from jax.experimental.pallas import tpu as pltpu
from jax.experimental.pallas import tpu_sc as plsc

_CHUNK = 128
_NBUF = 8


def _make_gather(num_rows: int, vocab: int, dim: int):
    info = plsc.get_sparse_core_info()
    nc, ns = info.num_cores, info.num_subcores
    nw = nc * ns
    assert num_rows % (nw * _CHUNK * _NBUF) == 0
    per_w = num_rows // nw
    n_chunks = per_w // _CHUNK
    n_outer = n_chunks // _NBUF

    mesh = plsc.VectorSubcoreMesh(core_axis_name="c", subcore_axis_name="s")

    @functools.partial(
        pl.kernel,
        mesh=mesh,
        compiler_params=pltpu.CompilerParams(use_tc_tiling_on_sc=False),
        out_type=jax.ShapeDtypeStruct((num_rows, dim), jnp.float32),
        scratch_types=[
            pltpu.VMEM((n_chunks, _CHUNK), jnp.int32),
            pltpu.VMEM((_NBUF, _CHUNK, dim), jnp.float32),
            pltpu.SemaphoreType.DMA((_NBUF,)),
            pltpu.SemaphoreType.DMA((_NBUF,)),
        ],
    )
    def emb(idx_hbm, tab_hbm, out_hbm, idx_v, rows_v, gsem, wsem):
        wid = lax.axis_index("s") * nc + lax.axis_index("c")
        base = wid * per_w

        pltpu.sync_copy(idx_hbm.at[wid], idx_v)

        def outer(g, carry):
            for b in range(_NBUF):
                c = g * _NBUF + b

                @pl.when(g > 0)
                def _wait_wb():
                    pltpu.make_async_copy(
                        rows_v.at[b],
                        out_hbm.at[pl.ds(base, _CHUNK)],
                        wsem.at[b],
                    ).wait()

                pltpu.async_copy(
                    tab_hbm.at[idx_v.at[c]],
                    rows_v.at[b],
                    gsem.at[b],
                )
            for b in range(_NBUF):
                c = g * _NBUF + b
                pltpu.make_async_copy(
                    tab_hbm.at[idx_v.at[c]],
                    rows_v.at[b],
                    gsem.at[b],
                ).wait()
                pltpu.make_async_copy(
                    rows_v.at[b],
                    out_hbm.at[pl.ds(base + c * _CHUNK, _CHUNK)],
                    wsem.at[b],
                ).start()
            return carry

        lax.fori_loop(0, n_outer, outer, 0)

        for b in range(_NBUF):
            pltpu.make_async_copy(
                rows_v.at[b],
                out_hbm.at[pl.ds(base, _CHUNK)],
                wsem.at[b],
            ).wait()

    return emb


def kernel(token_ids, embedding_matrix):
    b, h = token_ids.shape
    v, d = embedding_matrix.shape
    info = plsc.get_sparse_core_info()
    nw = info.num_cores * info.num_subcores
    flat = token_ids.reshape(nw, (b * h) // (nw * _CHUNK), _CHUNK).astype(jnp.int32)
    emb = _make_gather(b * h, v, d)
    out = emb(flat, embedding_matrix)
    return out.reshape(b, h, d)
